# baseline (speedup 1.0000x reference)
import jax
import jax.numpy as jnp
from jax import lax
from jax.experimental import pallas as pl
from jax.experimental.pallas import tpu as pltpu

E = 8
K = 2
D = 1024
F = 2048
N = 4096
S = N * K
B = 256
T = (S + E * (B - 1) + B - 1) // B
R = T * B

_NEG = -1e30
_GATE_BN = 512
_LANES = 128


def _gate_body(x_ref, wg_ref, sel_ref, w_ref):
    logits = jnp.dot(x_ref[...], wg_ref[...], preferred_element_type=jnp.float32)
    col = lax.broadcasted_iota(jnp.int32, logits.shape, 1)
    logits = jnp.where(col < E, logits, _NEG)
    m1 = jnp.max(logits, axis=1, keepdims=True)
    a1 = jnp.min(jnp.where(logits == m1, col, _LANES), axis=1, keepdims=True)
    l2 = jnp.where(col == a1, _NEG, logits)
    m2 = jnp.max(l2, axis=1, keepdims=True)
    a2 = jnp.min(jnp.where(l2 == m2, col, _LANES), axis=1, keepdims=True)
    e2 = jnp.exp(m2 - m1)
    w0 = 1.0 / (1.0 + e2)
    sel_ref[...] = jnp.where(col == 0, a1, jnp.where(col == 1, a2, 0)).astype(jnp.int32)
    w_ref[...] = jnp.where(col == 0, w0, jnp.where(col == 1, 1.0 - w0, 0.0))


def _gate(inputs, Wg):
    wg_pad = jnp.pad(Wg, ((0, 0), (0, _LANES - E)))
    sel_pad, w_pad = pl.pallas_call(
        _gate_body,
        grid=(N // _GATE_BN,),
        in_specs=[
            pl.BlockSpec((_GATE_BN, D), lambda i: (i, 0)),
            pl.BlockSpec((D, _LANES), lambda i: (0, 0)),
        ],
        out_specs=[
            pl.BlockSpec((_GATE_BN, _LANES), lambda i: (i, 0)),
            pl.BlockSpec((_GATE_BN, _LANES), lambda i: (i, 0)),
        ],
        out_shape=[
            jax.ShapeDtypeStruct((N, _LANES), jnp.int32),
            jax.ShapeDtypeStruct((N, _LANES), jnp.float32),
        ],
    )(inputs, wg_pad)
    return sel_pad[:, :K], w_pad[:, :K]


def _routing(sel):
    s = sel.reshape(-1)
    onehot = (s[:, None] == jnp.arange(E, dtype=jnp.int32)[None, :]).astype(jnp.int32)
    csum = jnp.cumsum(onehot, axis=0)
    rank = jnp.take_along_axis(csum, s[:, None], axis=1)[:, 0] - 1
    sizes = csum[-1]
    padded = ((sizes + B - 1) // B) * B
    bounds = jnp.cumsum(padded)
    starts = bounds - padded
    pos = starts[s] + rank
    tile_first = jnp.arange(T, dtype=jnp.int32) * B
    texp = jnp.minimum(
        jnp.searchsorted(bounds, tile_first, side="right"), E - 1
    ).astype(jnp.int32)
    return pos, texp


def _mlp_body(texp_ref, x_ref, w1_ref, w2_ref, wd_ref, y_ref):
    del texp_ref
    x = x_ref[...].astype(jnp.bfloat16)
    h = jnp.dot(x, w1_ref[0], preferred_element_type=jnp.float32)
    h = jax.nn.gelu(h)
    y = jnp.dot(h.astype(jnp.bfloat16), w2_ref[0], preferred_element_type=jnp.float32)
    y_ref[...] = y * wd_ref[0, 0][:, None]


def _grouped_mlp(texp, xd, W1b, W2b, wd3):
    grid_spec = pltpu.PrefetchScalarGridSpec(
        num_scalar_prefetch=1,
        grid=(T,),
        in_specs=[
            pl.BlockSpec((B, D), lambda i, t: (i, 0)),
            pl.BlockSpec((1, D, F), lambda i, t: (t[i], 0, 0)),
            pl.BlockSpec((1, F, D), lambda i, t: (t[i], 0, 0)),
            pl.BlockSpec((1, 1, B), lambda i, t: (i, 0, 0)),
        ],
        out_specs=pl.BlockSpec((B, D), lambda i, t: (i, 0)),
    )
    return pl.pallas_call(
        _mlp_body,
        grid_spec=grid_spec,
        out_shape=jax.ShapeDtypeStruct((R, D), jnp.float32),
    )(texp, xd, W1b, W2b, wd3)


def kernel(inputs, Wg, W1, W2):
    sel, w = _gate(inputs, Wg)
    pos, texp = _routing(sel)
    wflat = w.reshape(-1)
    wd = jnp.zeros((R,), jnp.float32).at[pos].set(wflat)

    xr = jnp.repeat(inputs, K, axis=0)
    xd = jnp.zeros((R, D), inputs.dtype).at[pos].set(xr)

    W1b = W1.astype(jnp.bfloat16)
    W2b = W2.astype(jnp.bfloat16)
    yd = _grouped_mlp(texp, xd, W1b, W2b, wd.reshape(T, 1, B))

    p2 = pos.reshape(N, K)
    return yd[p2[:, 0]] + yd[p2[:, 1]]


# speedup vs baseline: 1.3896x; 1.3896x over previous
import functools

import jax
import jax.numpy as jnp
from jax import lax
from jax.experimental import pallas as pl
from jax.experimental.pallas import tpu as pltpu
from jax.experimental.pallas import tpu_sc as plsc

E = 8
K = 2
D = 1024
F = 2048
N = 4096
S = N * K
B = 256
T = (S + E * (B - 1) + B - 1) // B
R = T * B

_NEG = -1e30
_GATE_BN = 512
_LANES = 128


def _gate_body(x_ref, wg_ref, sel_ref, w0_ref, w1_ref):
    logits = jnp.dot(x_ref[...], wg_ref[...], preferred_element_type=jnp.float32)
    col = lax.broadcasted_iota(jnp.int32, logits.shape, 1)
    logits = jnp.where(col < E, logits, _NEG)
    m1 = jnp.max(logits, axis=1, keepdims=True)
    a1 = jnp.min(jnp.where(logits == m1, col, _LANES), axis=1, keepdims=True)
    l2 = jnp.where(col == a1, _NEG, logits)
    m2 = jnp.max(l2, axis=1, keepdims=True)
    a2 = jnp.min(jnp.where(l2 == m2, col, _LANES), axis=1, keepdims=True)
    e2 = jnp.exp(m2 - m1)
    w0 = 1.0 / (1.0 + e2)
    sel_ref[...] = jnp.where(col == 0, a1, jnp.where(col == 1, a2, 0)).astype(jnp.int32)
    w0_ref[...] = jnp.broadcast_to(w0, w0_ref.shape)
    w1_ref[...] = jnp.broadcast_to(1.0 - w0, w1_ref.shape)


def _gate(inputs, Wg):
    wg_pad = jnp.pad(Wg, ((0, 0), (0, _LANES - E)))
    sel_pad, wb0, wb1 = pl.pallas_call(
        _gate_body,
        grid=(N // _GATE_BN,),
        in_specs=[
            pl.BlockSpec((_GATE_BN, D), lambda i: (i, 0)),
            pl.BlockSpec((D, _LANES), lambda i: (0, 0)),
        ],
        out_specs=[
            pl.BlockSpec((_GATE_BN, _LANES), lambda i: (i, 0)),
            pl.BlockSpec((_GATE_BN, _LANES), lambda i: (i, 0)),
            pl.BlockSpec((_GATE_BN, _LANES), lambda i: (i, 0)),
        ],
        out_shape=[
            jax.ShapeDtypeStruct((N, _LANES), jnp.int32),
            jax.ShapeDtypeStruct((N, _LANES), jnp.float32),
            jax.ShapeDtypeStruct((N, _LANES), jnp.float32),
        ],
    )(inputs, wg_pad)
    return sel_pad[:, :K], wb0, wb1


def _routing(sel):
    s = sel.reshape(-1)
    onehot = (s[:, None] == jnp.arange(E, dtype=jnp.int32)[None, :]).astype(jnp.int32)
    csum = jnp.cumsum(onehot, axis=0)
    rank = jnp.take_along_axis(csum, s[:, None], axis=1)[:, 0] - 1
    sizes = csum[-1]
    padded = ((sizes + B - 1) // B) * B
    bounds = jnp.cumsum(padded)
    starts = bounds - padded
    pos = starts[s] + rank
    tile_first = jnp.arange(T, dtype=jnp.int32) * B
    texp = jnp.minimum(
        jnp.searchsorted(bounds, tile_first, side="right"), E - 1
    ).astype(jnp.int32)
    return pos, texp


def _mlp_body(texp_ref, x_ref, w1_ref, w2_ref, wd_ref, y_ref):
    del texp_ref
    x = x_ref[...].astype(jnp.bfloat16)
    h = jnp.dot(x, w1_ref[0], preferred_element_type=jnp.float32)
    h = jax.nn.gelu(h)
    y = jnp.dot(h.astype(jnp.bfloat16), w2_ref[0], preferred_element_type=jnp.float32)
    y_ref[...] = y * wd_ref[:, :1]


def _grouped_mlp(texp, xd, W1b, W2b, wd):
    grid_spec = pltpu.PrefetchScalarGridSpec(
        num_scalar_prefetch=1,
        grid=(T,),
        in_specs=[
            pl.BlockSpec((B, D), lambda i, t: (i, 0)),
            pl.BlockSpec((1, D, F), lambda i, t: (t[i], 0, 0)),
            pl.BlockSpec((1, F, D), lambda i, t: (t[i], 0, 0)),
            pl.BlockSpec((B, _LANES), lambda i, t: (i, 0)),
        ],
        out_specs=pl.BlockSpec((B, D), lambda i, t: (i, 0)),
    )
    return pl.pallas_call(
        _mlp_body,
        grid_spec=grid_spec,
        out_shape=jax.ShapeDtypeStruct((R, D), jnp.float32),
    )(texp, xd, W1b, W2b, wd)


_MESH = plsc.VectorSubcoreMesh(core_axis_name="c", subcore_axis_name="s")
_NW = 32
_TPW = N // _NW
_CH = 32
_NCH = _TPW // _CH


@functools.partial(
    pl.kernel,
    mesh=_MESH,
    out_type=[
        jax.ShapeDtypeStruct((R, D), jnp.float32),
        jax.ShapeDtypeStruct((R, _LANES), jnp.float32),
    ],
    scratch_types=[
        pltpu.VMEM((_CH, D), jnp.float32),
        pltpu.VMEM((_CH, _LANES), jnp.float32),
        pltpu.VMEM((_CH, _LANES), jnp.float32),
        pltpu.VMEM((_CH,), jnp.int32),
        pltpu.VMEM((_CH,), jnp.int32),
        pltpu.SemaphoreType.DMA,
        pltpu.SemaphoreType.DMA,
        pltpu.SemaphoreType.DMA,
        pltpu.SemaphoreType.DMA,
    ],
)
def _dispatch_sc(x_hbm, wb0_hbm, wb1_hbm, p0_hbm, p1_hbm, xd_hbm, wd_hbm,
                 xbuf, wbuf0, wbuf1, idx0, idx1, sem0, sem1, sem2, sem3):
    wid = lax.axis_index("s") * 2 + lax.axis_index("c")
    for cc in range(_NCH):
        base = wid * _TPW + cc * _CH
        pltpu.sync_copy(x_hbm.at[pl.ds(base, _CH)], xbuf)
        pltpu.sync_copy(wb0_hbm.at[pl.ds(base, _CH)], wbuf0)
        pltpu.sync_copy(wb1_hbm.at[pl.ds(base, _CH)], wbuf1)
        pltpu.sync_copy(p0_hbm.at[pl.ds(base, _CH)], idx0)
        pltpu.sync_copy(p1_hbm.at[pl.ds(base, _CH)], idx1)
        c0 = pltpu.async_copy(xbuf, xd_hbm.at[idx0], sem0)
        c1 = pltpu.async_copy(xbuf, xd_hbm.at[idx1], sem1)
        c2 = pltpu.async_copy(wbuf0, wd_hbm.at[idx0], sem2)
        c3 = pltpu.async_copy(wbuf1, wd_hbm.at[idx1], sem3)
        c0.wait()
        c1.wait()
        c2.wait()
        c3.wait()


@functools.partial(
    pl.kernel,
    mesh=_MESH,
    out_type=jax.ShapeDtypeStruct((N, D), jnp.float32),
    scratch_types=[
        pltpu.VMEM((_CH, D), jnp.float32),
        pltpu.VMEM((_CH, D), jnp.float32),
        pltpu.VMEM((_CH, D), jnp.float32),
        pltpu.VMEM((_CH,), jnp.int32),
        pltpu.VMEM((_CH,), jnp.int32),
        pltpu.SemaphoreType.DMA,
        pltpu.SemaphoreType.DMA,
    ],
)
def _combine_sc(yd_hbm, p0_hbm, p1_hbm, out_hbm,
                buf0, buf1, obuf, idx0, idx1, sem0, sem1):
    wid = lax.axis_index("s") * 2 + lax.axis_index("c")
    for cc in range(_NCH):
        base = wid * _TPW + cc * _CH
        pltpu.sync_copy(p0_hbm.at[pl.ds(base, _CH)], idx0)
        pltpu.sync_copy(p1_hbm.at[pl.ds(base, _CH)], idx1)
        c0 = pltpu.async_copy(yd_hbm.at[idx0], buf0, sem0)
        c1 = pltpu.async_copy(yd_hbm.at[idx1], buf1, sem1)
        c0.wait()
        c1.wait()

        def row(r, carry):
            for c in range(D // 16):
                sl = pl.ds(c * 16, 16)
                obuf[r, sl] = buf0[r, sl] + buf1[r, sl]
            return carry

        lax.fori_loop(0, _CH, row, 0)
        pltpu.sync_copy(obuf, out_hbm.at[pl.ds(base, _CH)])


def kernel(inputs, Wg, W1, W2):
    sel, wb0, wb1 = _gate(inputs, Wg)
    pos, texp = _routing(sel)
    p2 = pos.reshape(N, K)
    p0, p1 = p2[:, 0], p2[:, 1]

    xd, wd = _dispatch_sc(inputs, wb0, wb1, p0, p1)

    W1b = W1.astype(jnp.bfloat16)
    W2b = W2.astype(jnp.bfloat16)
    yd = _grouped_mlp(texp, xd, W1b, W2b, wd)

    return _combine_sc(yd, p0, p1)


# speedup vs baseline: 1.6539x; 1.1902x over previous
import functools

import jax
import jax.numpy as jnp
from jax import lax
from jax.experimental import pallas as pl
from jax.experimental.pallas import tpu as pltpu
from jax.experimental.pallas import tpu_sc as plsc

E = 8
K = 2
D = 1024
F = 2048
N = 4096
S = N * K
B = 256
T = (S + E * (B - 1) + B - 1) // B
R = T * B

_NEG = -1e30
_GATE_BN = 512
_LANES = 128


def _gate_body(x_ref, wg_ref, sel_ref, w0_ref, w1_ref):
    logits = jnp.dot(x_ref[...], wg_ref[...], preferred_element_type=jnp.float32)
    col = lax.broadcasted_iota(jnp.int32, logits.shape, 1)
    logits = jnp.where(col < E, logits, _NEG)
    m1 = jnp.max(logits, axis=1, keepdims=True)
    a1 = jnp.min(jnp.where(logits == m1, col, _LANES), axis=1, keepdims=True)
    l2 = jnp.where(col == a1, _NEG, logits)
    m2 = jnp.max(l2, axis=1, keepdims=True)
    a2 = jnp.min(jnp.where(l2 == m2, col, _LANES), axis=1, keepdims=True)
    e2 = jnp.exp(m2 - m1)
    w0 = 1.0 / (1.0 + e2)
    sel_ref[...] = jnp.where(col == 0, a1, jnp.where(col == 1, a2, 0)).astype(jnp.int32)
    w0_ref[...] = jnp.broadcast_to(w0, w0_ref.shape)
    w1_ref[...] = jnp.broadcast_to(1.0 - w0, w1_ref.shape)


def _gate(inputs, Wg):
    wg_pad = jnp.pad(Wg, ((0, 0), (0, _LANES - E)))
    sel_pad, wb0, wb1 = pl.pallas_call(
        _gate_body,
        grid=(N // _GATE_BN,),
        in_specs=[
            pl.BlockSpec((_GATE_BN, D), lambda i: (i, 0)),
            pl.BlockSpec((D, _LANES), lambda i: (0, 0)),
        ],
        out_specs=[
            pl.BlockSpec((_GATE_BN, _LANES), lambda i: (i, 0)),
            pl.BlockSpec((_GATE_BN, _LANES), lambda i: (i, 0)),
            pl.BlockSpec((_GATE_BN, _LANES), lambda i: (i, 0)),
        ],
        out_shape=[
            jax.ShapeDtypeStruct((N, _LANES), jnp.int32),
            jax.ShapeDtypeStruct((N, _LANES), jnp.float32),
            jax.ShapeDtypeStruct((N, _LANES), jnp.float32),
        ],
    )(inputs, wg_pad)
    return sel_pad[:, :K], wb0, wb1


def _routing(sel):
    s = sel.reshape(-1)
    onehot = (s[:, None] == jnp.arange(E, dtype=jnp.int32)[None, :]).astype(jnp.int32)
    csum = jnp.cumsum(onehot, axis=0)
    rank = jnp.take_along_axis(csum, s[:, None], axis=1)[:, 0] - 1
    sizes = csum[-1]
    padded = ((sizes + B - 1) // B) * B
    bounds = jnp.cumsum(padded)
    starts = bounds - padded
    pos = starts[s] + rank
    tile_first = jnp.arange(T, dtype=jnp.int32) * B
    texp = jnp.minimum(
        jnp.searchsorted(bounds, tile_first, side="right"), E - 1
    ).astype(jnp.int32)
    return pos, texp


def _mlp_body(texp_ref, x_ref, w1_ref, w2_ref, wd_ref, y_ref):
    del texp_ref
    x = x_ref[...]
    h = jnp.dot(x, w1_ref[0], preferred_element_type=jnp.float32)
    h = jax.nn.gelu(h)
    y = jnp.dot(h, w2_ref[0], preferred_element_type=jnp.float32)
    y_ref[...] = y * wd_ref[:, :1]


def _grouped_mlp(texp, xd, W1b, W2b, wd):
    grid_spec = pltpu.PrefetchScalarGridSpec(
        num_scalar_prefetch=1,
        grid=(T,),
        in_specs=[
            pl.BlockSpec((B, D), lambda i, t: (i, 0)),
            pl.BlockSpec((1, D, F), lambda i, t: (t[i], 0, 0)),
            pl.BlockSpec((1, F, D), lambda i, t: (t[i], 0, 0)),
            pl.BlockSpec((B, _LANES), lambda i, t: (i, 0)),
        ],
        out_specs=pl.BlockSpec((B, D), lambda i, t: (i, 0)),
    )
    return pl.pallas_call(
        _mlp_body,
        grid_spec=grid_spec,
        out_shape=jax.ShapeDtypeStruct((R, D), jnp.float32),
    )(texp, xd, W1b, W2b, wd)


_MESH = plsc.VectorSubcoreMesh(core_axis_name="c", subcore_axis_name="s")
_NW = 32
_TPW = N // _NW
_CH = 32
_NCH = _TPW // _CH


@functools.partial(
    pl.kernel,
    mesh=_MESH,
    out_type=[
        jax.ShapeDtypeStruct((R, D), jnp.float32),
        jax.ShapeDtypeStruct((R, _LANES), jnp.float32),
    ],
    scratch_types=[
        pltpu.VMEM((_CH, D), jnp.float32),
        pltpu.VMEM((_CH, _LANES), jnp.float32),
        pltpu.VMEM((_CH, _LANES), jnp.float32),
        pltpu.VMEM((_CH,), jnp.int32),
        pltpu.VMEM((_CH,), jnp.int32),
        pltpu.SemaphoreType.DMA,
        pltpu.SemaphoreType.DMA,
        pltpu.SemaphoreType.DMA,
        pltpu.SemaphoreType.DMA,
    ],
)
def _dispatch_sc(x_hbm, wb0_hbm, wb1_hbm, p0_hbm, p1_hbm, xd_hbm, wd_hbm,
                 xbuf, wbuf0, wbuf1, idx0, idx1, sem0, sem1, sem2, sem3):
    wid = lax.axis_index("s") * 2 + lax.axis_index("c")
    for cc in range(_NCH):
        base = wid * _TPW + cc * _CH
        pltpu.sync_copy(x_hbm.at[pl.ds(base, _CH)], xbuf)
        pltpu.sync_copy(wb0_hbm.at[pl.ds(base, _CH)], wbuf0)
        pltpu.sync_copy(wb1_hbm.at[pl.ds(base, _CH)], wbuf1)
        pltpu.sync_copy(p0_hbm.at[pl.ds(base, _CH)], idx0)
        pltpu.sync_copy(p1_hbm.at[pl.ds(base, _CH)], idx1)
        c0 = pltpu.async_copy(xbuf, xd_hbm.at[idx0], sem0)
        c1 = pltpu.async_copy(xbuf, xd_hbm.at[idx1], sem1)
        c2 = pltpu.async_copy(wbuf0, wd_hbm.at[idx0], sem2)
        c3 = pltpu.async_copy(wbuf1, wd_hbm.at[idx1], sem3)
        c0.wait()
        c1.wait()
        c2.wait()
        c3.wait()


@functools.partial(
    pl.kernel,
    mesh=_MESH,
    out_type=jax.ShapeDtypeStruct((N, D), jnp.float32),
    scratch_types=[
        pltpu.VMEM((_CH, D), jnp.float32),
        pltpu.VMEM((_CH, D), jnp.float32),
        pltpu.VMEM((_CH, D), jnp.float32),
        pltpu.VMEM((_CH,), jnp.int32),
        pltpu.VMEM((_CH,), jnp.int32),
        pltpu.SemaphoreType.DMA,
        pltpu.SemaphoreType.DMA,
    ],
)
def _combine_sc(yd_hbm, p0_hbm, p1_hbm, out_hbm,
                buf0, buf1, obuf, idx0, idx1, sem0, sem1):
    wid = lax.axis_index("s") * 2 + lax.axis_index("c")
    for cc in range(_NCH):
        base = wid * _TPW + cc * _CH
        pltpu.sync_copy(p0_hbm.at[pl.ds(base, _CH)], idx0)
        pltpu.sync_copy(p1_hbm.at[pl.ds(base, _CH)], idx1)
        c0 = pltpu.async_copy(yd_hbm.at[idx0], buf0, sem0)
        c1 = pltpu.async_copy(yd_hbm.at[idx1], buf1, sem1)
        c0.wait()
        c1.wait()

        def row(r, carry):
            for c in range(D // 16):
                sl = pl.ds(c * 16, 16)
                obuf[r, sl] = buf0[r, sl] + buf1[r, sl]
            return carry

        lax.fori_loop(0, _CH, row, 0)
        pltpu.sync_copy(obuf, out_hbm.at[pl.ds(base, _CH)])


def kernel(inputs, Wg, W1, W2):
    sel, wb0, wb1 = _gate(inputs, Wg)
    pos, texp = _routing(sel)
    p2 = pos.reshape(N, K)
    p0, p1 = p2[:, 0], p2[:, 1]

    xd, wd = _dispatch_sc(inputs, wb0, wb1, p0, p1)

    yd = _grouped_mlp(texp, xd, W1, W2, wd)

    return _combine_sc(yd, p0, p1)


# speedup vs baseline: 1.7790x; 1.0756x over previous
import functools

import jax
import jax.numpy as jnp
from jax import lax
from jax.experimental import pallas as pl
from jax.experimental.pallas import tpu as pltpu
from jax.experimental.pallas import tpu_sc as plsc

E = 8
K = 2
D = 1024
F = 2048
N = 4096
S = N * K
B = 256
T = (S + E * (B - 1) + B - 1) // B
R = T * B

_NEG = -1e30
_GATE_BN = 512
_LANES = 128


def _gate_body(x_ref, wg_ref, sel_ref, w0_ref, w1_ref):
    logits = jnp.dot(x_ref[...], wg_ref[...], preferred_element_type=jnp.float32)
    col = lax.broadcasted_iota(jnp.int32, logits.shape, 1)
    logits = jnp.where(col < E, logits, _NEG)
    m1 = jnp.max(logits, axis=1, keepdims=True)
    a1 = jnp.min(jnp.where(logits == m1, col, _LANES), axis=1, keepdims=True)
    l2 = jnp.where(col == a1, _NEG, logits)
    m2 = jnp.max(l2, axis=1, keepdims=True)
    a2 = jnp.min(jnp.where(l2 == m2, col, _LANES), axis=1, keepdims=True)
    e2 = jnp.exp(m2 - m1)
    w0 = 1.0 / (1.0 + e2)
    sel_ref[...] = jnp.where(col == 0, a1, jnp.where(col == 1, a2, 0)).astype(jnp.int32)
    w0_ref[...] = jnp.broadcast_to(w0, w0_ref.shape)
    w1_ref[...] = jnp.broadcast_to(1.0 - w0, w1_ref.shape)


def _gate(inputs, Wg):
    wg_pad = jnp.pad(Wg, ((0, 0), (0, _LANES - E)))
    sel_pad, wb0, wb1 = pl.pallas_call(
        _gate_body,
        grid=(N // _GATE_BN,),
        in_specs=[
            pl.BlockSpec((_GATE_BN, D), lambda i: (i, 0)),
            pl.BlockSpec((D, _LANES), lambda i: (0, 0)),
        ],
        out_specs=[
            pl.BlockSpec((_GATE_BN, _LANES), lambda i: (i, 0)),
            pl.BlockSpec((_GATE_BN, _LANES), lambda i: (i, 0)),
            pl.BlockSpec((_GATE_BN, _LANES), lambda i: (i, 0)),
        ],
        out_shape=[
            jax.ShapeDtypeStruct((N, _LANES), jnp.int32),
            jax.ShapeDtypeStruct((N, _LANES), jnp.float32),
            jax.ShapeDtypeStruct((N, _LANES), jnp.float32),
        ],
    )(inputs, wg_pad)
    return sel_pad[:, :K], wb0, wb1


def _routing(sel):
    s = sel.reshape(-1)
    onehot = (s[:, None] == jnp.arange(E, dtype=jnp.int32)[None, :]).astype(jnp.int32)
    csum = jnp.cumsum(onehot, axis=0)
    rank = jnp.take_along_axis(csum, s[:, None], axis=1)[:, 0] - 1
    sizes = csum[-1]
    padded = ((sizes + B - 1) // B) * B
    bounds = jnp.cumsum(padded)
    starts = bounds - padded
    pos = starts[s] + rank
    tile_first = jnp.arange(T, dtype=jnp.int32) * B
    texp = jnp.minimum(
        jnp.searchsorted(bounds, tile_first, side="right"), E - 1
    ).astype(jnp.int32)
    a_tiles = bounds[-1] // B
    ii = jnp.arange(T, dtype=jnp.int32)
    xblk = jnp.minimum(ii, a_tiles - 1)
    act = (ii < a_tiles).astype(jnp.int32)
    sa = jnp.stack([texp[xblk], act, xblk, jnp.zeros_like(ii)], axis=1)
    return pos, sa


def _mlp_body(sa_ref, x_ref, w1_ref, w2_ref, wd_ref, y_ref):
    i = pl.program_id(0)

    @pl.when(sa_ref[i, 1] != 0)
    def _():
        h = jnp.dot(x_ref[...], w1_ref[0], preferred_element_type=jnp.float32)
        h = jax.nn.gelu(h)
        y = jnp.dot(h, w2_ref[0], preferred_element_type=jnp.float32)
        y_ref[...] = y * wd_ref[:, :1]


def _grouped_mlp(sa, xd, W1, W2, wd):
    grid_spec = pltpu.PrefetchScalarGridSpec(
        num_scalar_prefetch=1,
        grid=(T,),
        in_specs=[
            pl.BlockSpec((B, D), lambda i, t: (t[i, 2], 0)),
            pl.BlockSpec((1, D, F), lambda i, t: (t[i, 0], 0, 0)),
            pl.BlockSpec((1, F, D), lambda i, t: (t[i, 0], 0, 0)),
            pl.BlockSpec((B, _LANES), lambda i, t: (t[i, 2], 0)),
        ],
        out_specs=pl.BlockSpec((B, D), lambda i, t: (t[i, 2], 0)),
    )
    return pl.pallas_call(
        _mlp_body,
        grid_spec=grid_spec,
        out_shape=jax.ShapeDtypeStruct((R, D), jnp.float32),
    )(sa, xd, W1, W2, wd)


_MESH = plsc.VectorSubcoreMesh(core_axis_name="c", subcore_axis_name="s")
_NW = 32
_TPW = N // _NW
_DCH = 32
_DNCH = _TPW // _DCH
_CCH = 16
_CNCH = _TPW // _CCH


@functools.partial(
    pl.kernel,
    mesh=_MESH,
    out_type=[
        jax.ShapeDtypeStruct((R, D), jnp.float32),
        jax.ShapeDtypeStruct((R, _LANES), jnp.float32),
    ],
    scratch_types=[
        pltpu.VMEM((2, _DCH, D), jnp.float32),
        pltpu.VMEM((_TPW, _LANES), jnp.float32),
        pltpu.VMEM((_TPW, _LANES), jnp.float32),
        pltpu.VMEM((_DNCH, _DCH), jnp.int32),
        pltpu.VMEM((_DNCH, _DCH), jnp.int32),
        pltpu.SemaphoreType.DMA,
        pltpu.SemaphoreType.DMA,
        pltpu.SemaphoreType.DMA,
        pltpu.SemaphoreType.DMA,
        pltpu.SemaphoreType.DMA,
        pltpu.SemaphoreType.DMA,
        pltpu.SemaphoreType.DMA,
        pltpu.SemaphoreType.DMA,
        pltpu.SemaphoreType.DMA,
        pltpu.SemaphoreType.DMA,
    ],
)
def _dispatch_sc(x_hbm, wb0_hbm, wb1_hbm, p0_hbm, p1_hbm, xd_hbm, wd_hbm,
                 xbuf, wbuf0, wbuf1, idx0b, idx1b, ls0, ls1, *ss):
    wid = lax.axis_index("s") * 2 + lax.axis_index("c")
    tok0 = wid * _TPW
    pltpu.sync_copy(wb0_hbm.at[pl.ds(tok0, _TPW)], wbuf0)
    pltpu.sync_copy(wb1_hbm.at[pl.ds(tok0, _TPW)], wbuf1)
    pltpu.sync_copy(p0_hbm.at[pl.ds(wid * _DNCH, _DNCH)], idx0b)
    pltpu.sync_copy(p1_hbm.at[pl.ds(wid * _DNCH, _DNCH)], idx1b)
    lsems = (ls0, ls1)
    loads = [None, None]
    scat = [None] * _DNCH
    loads[0] = pltpu.async_copy(x_hbm.at[pl.ds(tok0, _DCH)], xbuf.at[0], lsems[0])
    for cc in range(_DNCH):
        slot = cc % 2
        loads[slot].wait()
        sg = ss[4 * slot:4 * slot + 4]
        scat[cc] = [
            pltpu.async_copy(xbuf.at[slot], xd_hbm.at[idx0b.at[cc]], sg[0]),
            pltpu.async_copy(xbuf.at[slot], xd_hbm.at[idx1b.at[cc]], sg[1]),
            pltpu.async_copy(wbuf0.at[pl.ds(cc * _DCH, _DCH)],
                             wd_hbm.at[idx0b.at[cc]], sg[2]),
            pltpu.async_copy(wbuf1.at[pl.ds(cc * _DCH, _DCH)],
                             wd_hbm.at[idx1b.at[cc]], sg[3]),
        ]
        if cc + 1 < _DNCH:
            if cc >= 1:
                for c in scat[cc - 1]:
                    c.wait()
            loads[1 - slot] = pltpu.async_copy(
                x_hbm.at[pl.ds(tok0 + (cc + 1) * _DCH, _DCH)],
                xbuf.at[1 - slot], lsems[1 - slot])
    for cc in (_DNCH - 2, _DNCH - 1):
        for c in scat[cc]:
            c.wait()


@functools.partial(
    pl.kernel,
    mesh=_MESH,
    out_type=jax.ShapeDtypeStruct((N, D), jnp.float32),
    scratch_types=[
        pltpu.VMEM((2, _CCH, D), jnp.float32),
        pltpu.VMEM((2, _CCH, D), jnp.float32),
        pltpu.VMEM((2, _CCH, D), jnp.float32),
        pltpu.VMEM((_CNCH, _CCH), jnp.int32),
        pltpu.VMEM((_CNCH, _CCH), jnp.int32),
        pltpu.SemaphoreType.DMA,
        pltpu.SemaphoreType.DMA,
        pltpu.SemaphoreType.DMA,
        pltpu.SemaphoreType.DMA,
        pltpu.SemaphoreType.DMA,
        pltpu.SemaphoreType.DMA,
    ],
)
def _combine_sc(yd_hbm, p0_hbm, p1_hbm, out_hbm,
                b0, b1, ob, i0b, i1b, g0, g1, g2, g3, ws0, ws1):
    wid = lax.axis_index("s") * 2 + lax.axis_index("c")
    tok0 = wid * _TPW
    pltpu.sync_copy(p0_hbm.at[pl.ds(wid * _CNCH, _CNCH)], i0b)
    pltpu.sync_copy(p1_hbm.at[pl.ds(wid * _CNCH, _CNCH)], i1b)
    gs = ((g0, g1), (g2, g3))
    wsems = (ws0, ws1)
    gat = [None] * _CNCH
    wr = [None] * _CNCH
    gat[0] = (pltpu.async_copy(yd_hbm.at[i0b.at[0]], b0.at[0], gs[0][0]),
              pltpu.async_copy(yd_hbm.at[i1b.at[0]], b1.at[0], gs[0][1]))
    for cc in range(_CNCH):
        slot = cc % 2
        for c in gat[cc]:
            c.wait()
        if cc + 1 < _CNCH:
            gat[cc + 1] = (
                pltpu.async_copy(yd_hbm.at[i0b.at[cc + 1]], b0.at[1 - slot],
                                 gs[1 - slot][0]),
                pltpu.async_copy(yd_hbm.at[i1b.at[cc + 1]], b1.at[1 - slot],
                                 gs[1 - slot][1]),
            )
        if cc >= 2:
            wr[cc - 2].wait()

        def row(r, carry):
            for c in range(D // 16):
                sl = pl.ds(c * 16, 16)
                ob[slot, r, sl] = b0[slot, r, sl] + b1[slot, r, sl]
            return carry

        lax.fori_loop(0, _CCH, row, 0)
        wr[cc] = pltpu.async_copy(
            ob.at[slot], out_hbm.at[pl.ds(tok0 + cc * _CCH, _CCH)], wsems[slot])
    wr[_CNCH - 2].wait()
    wr[_CNCH - 1].wait()


def kernel(inputs, Wg, W1, W2):
    sel, wb0, wb1 = _gate(inputs, Wg)
    pos, sa = _routing(sel)
    p2 = pos.reshape(N, K)
    p0, p1 = p2[:, 0], p2[:, 1]

    xd, wd = _dispatch_sc(inputs, wb0, wb1,
                          p0.reshape(N // _DCH, _DCH), p1.reshape(N // _DCH, _DCH))

    yd = _grouped_mlp(sa, xd, W1, W2, wd)

    return _combine_sc(yd, p0.reshape(N // _CCH, _CCH), p1.reshape(N // _CCH, _CCH))


# speedup vs baseline: 1.9390x; 1.0899x over previous
import functools

import jax
import jax.numpy as jnp
from jax import lax
from jax.experimental import pallas as pl
from jax.experimental.pallas import tpu as pltpu
from jax.experimental.pallas import tpu_sc as plsc

E = 8
K = 2
D = 1024
F = 2048
N = 4096
S = N * K
B = 512
T = (S + E * (B - 1) + B - 1) // B
R = T * B

_NEG = -1e30
_GATE_BN = 512
_LANES = 128


def _gate_body(x_ref, wg_ref, sel_ref, w0_ref, w1_ref):
    logits = jnp.dot(x_ref[...], wg_ref[...], preferred_element_type=jnp.float32)
    col = lax.broadcasted_iota(jnp.int32, logits.shape, 1)
    logits = jnp.where(col < E, logits, _NEG)
    m1 = jnp.max(logits, axis=1, keepdims=True)
    a1 = jnp.min(jnp.where(logits == m1, col, _LANES), axis=1, keepdims=True)
    l2 = jnp.where(col == a1, _NEG, logits)
    m2 = jnp.max(l2, axis=1, keepdims=True)
    a2 = jnp.min(jnp.where(l2 == m2, col, _LANES), axis=1, keepdims=True)
    e2 = jnp.exp(m2 - m1)
    w0 = 1.0 / (1.0 + e2)
    sel_ref[...] = jnp.where(col == 0, a1, jnp.where(col == 1, a2, 0)).astype(jnp.int32)
    w0_ref[...] = jnp.broadcast_to(w0, w0_ref.shape)
    w1_ref[...] = jnp.broadcast_to(1.0 - w0, w1_ref.shape)


def _gate(inputs, Wg):
    wg_pad = jnp.pad(Wg, ((0, 0), (0, _LANES - E)))
    sel_pad, wb0, wb1 = pl.pallas_call(
        _gate_body,
        grid=(N // _GATE_BN,),
        in_specs=[
            pl.BlockSpec((_GATE_BN, D), lambda i: (i, 0)),
            pl.BlockSpec((D, _LANES), lambda i: (0, 0)),
        ],
        out_specs=[
            pl.BlockSpec((_GATE_BN, _LANES), lambda i: (i, 0)),
            pl.BlockSpec((_GATE_BN, _LANES), lambda i: (i, 0)),
            pl.BlockSpec((_GATE_BN, _LANES), lambda i: (i, 0)),
        ],
        out_shape=[
            jax.ShapeDtypeStruct((N, _LANES), jnp.int32),
            jax.ShapeDtypeStruct((N, _LANES), jnp.float32),
            jax.ShapeDtypeStruct((N, _LANES), jnp.float32),
        ],
    )(inputs, wg_pad)
    return sel_pad[:, :K], wb0, wb1


def _routing(sel):
    s = sel.reshape(-1)
    onehot = (s[:, None] == jnp.arange(E, dtype=jnp.int32)[None, :]).astype(jnp.int32)
    csum = jnp.cumsum(onehot, axis=0)
    rank = jnp.take_along_axis(csum, s[:, None], axis=1)[:, 0] - 1
    sizes = csum[-1]
    padded = ((sizes + B - 1) // B) * B
    bounds = jnp.cumsum(padded)
    starts = bounds - padded
    pos = starts[s] + rank
    tile_first = jnp.arange(T, dtype=jnp.int32) * B
    texp = jnp.minimum(
        jnp.searchsorted(bounds, tile_first, side="right"), E - 1
    ).astype(jnp.int32)
    a_tiles = bounds[-1] // B
    ii = jnp.arange(T, dtype=jnp.int32)
    xblk = jnp.minimum(ii, a_tiles - 1)
    act = (ii < a_tiles).astype(jnp.int32)
    sa = jnp.stack([texp[xblk], act, xblk, jnp.zeros_like(ii)], axis=1)
    return pos, sa


def _mlp_body(sa_ref, x_ref, w1_ref, w2_ref, wd_ref, y_ref):
    i = pl.program_id(0)

    @pl.when(sa_ref[i, 1] != 0)
    def _():
        h = jnp.dot(x_ref[...], w1_ref[0], preferred_element_type=jnp.float32)
        h = jax.nn.gelu(h)
        y = jnp.dot(h, w2_ref[0], preferred_element_type=jnp.float32)
        y_ref[...] = y * wd_ref[:, :1]


def _grouped_mlp(sa, xd, W1, W2, wd):
    grid_spec = pltpu.PrefetchScalarGridSpec(
        num_scalar_prefetch=1,
        grid=(T,),
        in_specs=[
            pl.BlockSpec((B, D), lambda i, t: (t[i, 2], 0)),
            pl.BlockSpec((1, D, F), lambda i, t: (t[i, 0], 0, 0)),
            pl.BlockSpec((1, F, D), lambda i, t: (t[i, 0], 0, 0)),
            pl.BlockSpec((B, _LANES), lambda i, t: (t[i, 2], 0)),
        ],
        out_specs=pl.BlockSpec((B, D), lambda i, t: (t[i, 2], 0)),
    )
    return pl.pallas_call(
        _mlp_body,
        grid_spec=grid_spec,
        out_shape=jax.ShapeDtypeStruct((R, D), jnp.float32),
    )(sa, xd, W1, W2, wd)


_MESH = plsc.VectorSubcoreMesh(core_axis_name="c", subcore_axis_name="s")
_NW = 32
_TPW = N // _NW
_DCH = 32
_DNCH = _TPW // _DCH
_CCH = 16
_CNCH = _TPW // _CCH


@functools.partial(
    pl.kernel,
    mesh=_MESH,
    out_type=[
        jax.ShapeDtypeStruct((R, D), jnp.float32),
        jax.ShapeDtypeStruct((R, _LANES), jnp.float32),
    ],
    scratch_types=[
        pltpu.VMEM((2, _DCH, D), jnp.float32),
        pltpu.VMEM((_TPW, _LANES), jnp.float32),
        pltpu.VMEM((_TPW, _LANES), jnp.float32),
        pltpu.VMEM((_DNCH, _DCH), jnp.int32),
        pltpu.VMEM((_DNCH, _DCH), jnp.int32),
        pltpu.SemaphoreType.DMA,
        pltpu.SemaphoreType.DMA,
        pltpu.SemaphoreType.DMA,
        pltpu.SemaphoreType.DMA,
        pltpu.SemaphoreType.DMA,
        pltpu.SemaphoreType.DMA,
        pltpu.SemaphoreType.DMA,
        pltpu.SemaphoreType.DMA,
        pltpu.SemaphoreType.DMA,
        pltpu.SemaphoreType.DMA,
    ],
)
def _dispatch_sc(x_hbm, wb0_hbm, wb1_hbm, p0_hbm, p1_hbm, xd_hbm, wd_hbm,
                 xbuf, wbuf0, wbuf1, idx0b, idx1b, ls0, ls1, *ss):
    wid = lax.axis_index("s") * 2 + lax.axis_index("c")
    tok0 = wid * _TPW
    pltpu.sync_copy(wb0_hbm.at[pl.ds(tok0, _TPW)], wbuf0)
    pltpu.sync_copy(wb1_hbm.at[pl.ds(tok0, _TPW)], wbuf1)
    pltpu.sync_copy(p0_hbm.at[pl.ds(wid * _DNCH, _DNCH)], idx0b)
    pltpu.sync_copy(p1_hbm.at[pl.ds(wid * _DNCH, _DNCH)], idx1b)
    lsems = (ls0, ls1)
    loads = [None, None]
    scat = [None] * _DNCH
    loads[0] = pltpu.async_copy(x_hbm.at[pl.ds(tok0, _DCH)], xbuf.at[0], lsems[0])
    for cc in range(_DNCH):
        slot = cc % 2
        loads[slot].wait()
        sg = ss[4 * slot:4 * slot + 4]
        scat[cc] = [
            pltpu.async_copy(xbuf.at[slot], xd_hbm.at[idx0b.at[cc]], sg[0]),
            pltpu.async_copy(xbuf.at[slot], xd_hbm.at[idx1b.at[cc]], sg[1]),
            pltpu.async_copy(wbuf0.at[pl.ds(cc * _DCH, _DCH)],
                             wd_hbm.at[idx0b.at[cc]], sg[2]),
            pltpu.async_copy(wbuf1.at[pl.ds(cc * _DCH, _DCH)],
                             wd_hbm.at[idx1b.at[cc]], sg[3]),
        ]
        if cc + 1 < _DNCH:
            if cc >= 1:
                for c in scat[cc - 1]:
                    c.wait()
            loads[1 - slot] = pltpu.async_copy(
                x_hbm.at[pl.ds(tok0 + (cc + 1) * _DCH, _DCH)],
                xbuf.at[1 - slot], lsems[1 - slot])
    for cc in (_DNCH - 2, _DNCH - 1):
        for c in scat[cc]:
            c.wait()


@functools.partial(
    pl.kernel,
    mesh=_MESH,
    out_type=jax.ShapeDtypeStruct((N, D), jnp.float32),
    scratch_types=[
        pltpu.VMEM((2, _CCH, D), jnp.float32),
        pltpu.VMEM((2, _CCH, D), jnp.float32),
        pltpu.VMEM((2, _CCH, D), jnp.float32),
        pltpu.VMEM((_CNCH, _CCH), jnp.int32),
        pltpu.VMEM((_CNCH, _CCH), jnp.int32),
        pltpu.SemaphoreType.DMA,
        pltpu.SemaphoreType.DMA,
        pltpu.SemaphoreType.DMA,
        pltpu.SemaphoreType.DMA,
        pltpu.SemaphoreType.DMA,
        pltpu.SemaphoreType.DMA,
    ],
)
def _combine_sc(yd_hbm, p0_hbm, p1_hbm, out_hbm,
                b0, b1, ob, i0b, i1b, g0, g1, g2, g3, ws0, ws1):
    wid = lax.axis_index("s") * 2 + lax.axis_index("c")
    tok0 = wid * _TPW
    pltpu.sync_copy(p0_hbm.at[pl.ds(wid * _CNCH, _CNCH)], i0b)
    pltpu.sync_copy(p1_hbm.at[pl.ds(wid * _CNCH, _CNCH)], i1b)
    gs = ((g0, g1), (g2, g3))
    wsems = (ws0, ws1)
    gat = [None] * _CNCH
    wr = [None] * _CNCH
    gat[0] = (pltpu.async_copy(yd_hbm.at[i0b.at[0]], b0.at[0], gs[0][0]),
              pltpu.async_copy(yd_hbm.at[i1b.at[0]], b1.at[0], gs[0][1]))
    for cc in range(_CNCH):
        slot = cc % 2
        for c in gat[cc]:
            c.wait()
        if cc + 1 < _CNCH:
            gat[cc + 1] = (
                pltpu.async_copy(yd_hbm.at[i0b.at[cc + 1]], b0.at[1 - slot],
                                 gs[1 - slot][0]),
                pltpu.async_copy(yd_hbm.at[i1b.at[cc + 1]], b1.at[1 - slot],
                                 gs[1 - slot][1]),
            )
        if cc >= 2:
            wr[cc - 2].wait()

        def row(r, carry):
            for c in range(D // 16):
                sl = pl.ds(c * 16, 16)
                ob[slot, r, sl] = b0[slot, r, sl] + b1[slot, r, sl]
            return carry

        lax.fori_loop(0, _CCH, row, 0)
        wr[cc] = pltpu.async_copy(
            ob.at[slot], out_hbm.at[pl.ds(tok0 + cc * _CCH, _CCH)], wsems[slot])
    wr[_CNCH - 2].wait()
    wr[_CNCH - 1].wait()


def kernel(inputs, Wg, W1, W2):
    sel, wb0, wb1 = _gate(inputs, Wg)
    pos, sa = _routing(sel)
    p2 = pos.reshape(N, K)
    p0, p1 = p2[:, 0], p2[:, 1]

    xd, wd = _dispatch_sc(inputs, wb0, wb1,
                          p0.reshape(N // _DCH, _DCH), p1.reshape(N // _DCH, _DCH))

    yd = _grouped_mlp(sa, xd, W1, W2, wd)

    return _combine_sc(yd, p0.reshape(N // _CCH, _CCH), p1.reshape(N // _CCH, _CCH))


# speedup vs baseline: 2.0110x; 1.0372x over previous
import functools

import jax
import jax.numpy as jnp
from jax import lax
from jax.experimental import pallas as pl
from jax.experimental.pallas import tpu as pltpu
from jax.experimental.pallas import tpu_sc as plsc

E = 8
K = 2
D = 1024
F = 2048
N = 4096
S = N * K
B = 1024
T = (S + E * (B - 1) + B - 1) // B
R = T * B

_NEG = -1e30
_GATE_BN = 512
_LANES = 128


def _gate_body(x_ref, wg_ref, sel_ref, w0_ref, w1_ref):
    logits = jnp.dot(x_ref[...], wg_ref[...], preferred_element_type=jnp.float32)
    col = lax.broadcasted_iota(jnp.int32, logits.shape, 1)
    logits = jnp.where(col < E, logits, _NEG)
    m1 = jnp.max(logits, axis=1, keepdims=True)
    a1 = jnp.min(jnp.where(logits == m1, col, _LANES), axis=1, keepdims=True)
    l2 = jnp.where(col == a1, _NEG, logits)
    m2 = jnp.max(l2, axis=1, keepdims=True)
    a2 = jnp.min(jnp.where(l2 == m2, col, _LANES), axis=1, keepdims=True)
    e2 = jnp.exp(m2 - m1)
    w0 = 1.0 / (1.0 + e2)
    sel_ref[...] = jnp.where(col == 0, a1, jnp.where(col == 1, a2, 0)).astype(jnp.int32)
    w0_ref[...] = jnp.broadcast_to(w0, w0_ref.shape)
    w1_ref[...] = jnp.broadcast_to(1.0 - w0, w1_ref.shape)


def _gate(inputs, Wg):
    wg_pad = jnp.pad(Wg, ((0, 0), (0, _LANES - E)))
    sel_pad, wb0, wb1 = pl.pallas_call(
        _gate_body,
        grid=(N // _GATE_BN,),
        in_specs=[
            pl.BlockSpec((_GATE_BN, D), lambda i: (i, 0)),
            pl.BlockSpec((D, _LANES), lambda i: (0, 0)),
        ],
        out_specs=[
            pl.BlockSpec((_GATE_BN, _LANES), lambda i: (i, 0)),
            pl.BlockSpec((_GATE_BN, _LANES), lambda i: (i, 0)),
            pl.BlockSpec((_GATE_BN, _LANES), lambda i: (i, 0)),
        ],
        out_shape=[
            jax.ShapeDtypeStruct((N, _LANES), jnp.int32),
            jax.ShapeDtypeStruct((N, _LANES), jnp.float32),
            jax.ShapeDtypeStruct((N, _LANES), jnp.float32),
        ],
    )(inputs, wg_pad)
    return sel_pad[:, :K], wb0, wb1


def _routing(sel):
    s = sel.reshape(-1)
    onehot = (s[:, None] == jnp.arange(E, dtype=jnp.int32)[None, :]).astype(jnp.int32)
    csum = jnp.cumsum(onehot, axis=0)
    rank = jnp.take_along_axis(csum, s[:, None], axis=1)[:, 0] - 1
    sizes = csum[-1]
    padded = ((sizes + B - 1) // B) * B
    bounds = jnp.cumsum(padded)
    starts = bounds - padded
    pos = starts[s] + rank
    tile_first = jnp.arange(T, dtype=jnp.int32) * B
    texp = jnp.minimum(
        jnp.searchsorted(bounds, tile_first, side="right"), E - 1
    ).astype(jnp.int32)
    a_tiles = bounds[-1] // B
    ii = jnp.arange(T, dtype=jnp.int32)
    xblk = jnp.minimum(ii, a_tiles - 1)
    act = (ii < a_tiles).astype(jnp.int32)
    sa = jnp.stack([texp[xblk], act, xblk, jnp.zeros_like(ii)], axis=1)
    return pos, sa


def _mlp_body(sa_ref, x_ref, w1_ref, w2_ref, wd_ref, y_ref):
    i = pl.program_id(0)

    @pl.when(sa_ref[i, 1] != 0)
    def _():
        h = jnp.dot(x_ref[...], w1_ref[0], preferred_element_type=jnp.float32)
        h = jax.nn.gelu(h)
        y = jnp.dot(h, w2_ref[0], preferred_element_type=jnp.float32)
        y_ref[...] = y * wd_ref[:, :1]


def _grouped_mlp(sa, xd, W1, W2, wd):
    grid_spec = pltpu.PrefetchScalarGridSpec(
        num_scalar_prefetch=1,
        grid=(T,),
        in_specs=[
            pl.BlockSpec((B, D), lambda i, t: (t[i, 2], 0)),
            pl.BlockSpec((1, D, F), lambda i, t: (t[i, 0], 0, 0)),
            pl.BlockSpec((1, F, D), lambda i, t: (t[i, 0], 0, 0)),
            pl.BlockSpec((B, _LANES), lambda i, t: (t[i, 2], 0)),
        ],
        out_specs=pl.BlockSpec((B, D), lambda i, t: (t[i, 2], 0)),
    )
    return pl.pallas_call(
        _mlp_body,
        grid_spec=grid_spec,
        out_shape=jax.ShapeDtypeStruct((R, D), jnp.float32),
    )(sa, xd, W1, W2, wd)


_MESH = plsc.VectorSubcoreMesh(core_axis_name="c", subcore_axis_name="s")
_NW = 32
_TPW = N // _NW
_DCH = 32
_DNCH = _TPW // _DCH
_CCH = 16
_CNCH = _TPW // _CCH


@functools.partial(
    pl.kernel,
    mesh=_MESH,
    out_type=[
        jax.ShapeDtypeStruct((R, D), jnp.float32),
        jax.ShapeDtypeStruct((R, _LANES), jnp.float32),
    ],
    scratch_types=[
        pltpu.VMEM((2, _DCH, D), jnp.float32),
        pltpu.VMEM((_TPW, _LANES), jnp.float32),
        pltpu.VMEM((_TPW, _LANES), jnp.float32),
        pltpu.VMEM((_DNCH, _DCH), jnp.int32),
        pltpu.VMEM((_DNCH, _DCH), jnp.int32),
        pltpu.SemaphoreType.DMA,
        pltpu.SemaphoreType.DMA,
        pltpu.SemaphoreType.DMA,
        pltpu.SemaphoreType.DMA,
        pltpu.SemaphoreType.DMA,
        pltpu.SemaphoreType.DMA,
        pltpu.SemaphoreType.DMA,
        pltpu.SemaphoreType.DMA,
        pltpu.SemaphoreType.DMA,
        pltpu.SemaphoreType.DMA,
    ],
)
def _dispatch_sc(x_hbm, wb0_hbm, wb1_hbm, p0_hbm, p1_hbm, xd_hbm, wd_hbm,
                 xbuf, wbuf0, wbuf1, idx0b, idx1b, ls0, ls1, *ss):
    wid = lax.axis_index("s") * 2 + lax.axis_index("c")
    tok0 = wid * _TPW
    pltpu.sync_copy(wb0_hbm.at[pl.ds(tok0, _TPW)], wbuf0)
    pltpu.sync_copy(wb1_hbm.at[pl.ds(tok0, _TPW)], wbuf1)
    pltpu.sync_copy(p0_hbm.at[pl.ds(wid * _DNCH, _DNCH)], idx0b)
    pltpu.sync_copy(p1_hbm.at[pl.ds(wid * _DNCH, _DNCH)], idx1b)
    lsems = (ls0, ls1)
    loads = [None, None]
    scat = [None] * _DNCH
    loads[0] = pltpu.async_copy(x_hbm.at[pl.ds(tok0, _DCH)], xbuf.at[0], lsems[0])
    for cc in range(_DNCH):
        slot = cc % 2
        loads[slot].wait()
        sg = ss[4 * slot:4 * slot + 4]
        scat[cc] = [
            pltpu.async_copy(xbuf.at[slot], xd_hbm.at[idx0b.at[cc]], sg[0]),
            pltpu.async_copy(xbuf.at[slot], xd_hbm.at[idx1b.at[cc]], sg[1]),
            pltpu.async_copy(wbuf0.at[pl.ds(cc * _DCH, _DCH)],
                             wd_hbm.at[idx0b.at[cc]], sg[2]),
            pltpu.async_copy(wbuf1.at[pl.ds(cc * _DCH, _DCH)],
                             wd_hbm.at[idx1b.at[cc]], sg[3]),
        ]
        if cc + 1 < _DNCH:
            if cc >= 1:
                for c in scat[cc - 1]:
                    c.wait()
            loads[1 - slot] = pltpu.async_copy(
                x_hbm.at[pl.ds(tok0 + (cc + 1) * _DCH, _DCH)],
                xbuf.at[1 - slot], lsems[1 - slot])
    for cc in (_DNCH - 2, _DNCH - 1):
        for c in scat[cc]:
            c.wait()


@functools.partial(
    pl.kernel,
    mesh=_MESH,
    out_type=jax.ShapeDtypeStruct((N, D), jnp.float32),
    scratch_types=[
        pltpu.VMEM((2, _CCH, D), jnp.float32),
        pltpu.VMEM((2, _CCH, D), jnp.float32),
        pltpu.VMEM((2, _CCH, D), jnp.float32),
        pltpu.VMEM((_CNCH, _CCH), jnp.int32),
        pltpu.VMEM((_CNCH, _CCH), jnp.int32),
        pltpu.SemaphoreType.DMA,
        pltpu.SemaphoreType.DMA,
        pltpu.SemaphoreType.DMA,
        pltpu.SemaphoreType.DMA,
        pltpu.SemaphoreType.DMA,
        pltpu.SemaphoreType.DMA,
    ],
)
def _combine_sc(yd_hbm, p0_hbm, p1_hbm, out_hbm,
                b0, b1, ob, i0b, i1b, g0, g1, g2, g3, ws0, ws1):
    wid = lax.axis_index("s") * 2 + lax.axis_index("c")
    tok0 = wid * _TPW
    pltpu.sync_copy(p0_hbm.at[pl.ds(wid * _CNCH, _CNCH)], i0b)
    pltpu.sync_copy(p1_hbm.at[pl.ds(wid * _CNCH, _CNCH)], i1b)
    gs = ((g0, g1), (g2, g3))
    wsems = (ws0, ws1)
    gat = [None] * _CNCH
    wr = [None] * _CNCH
    gat[0] = (pltpu.async_copy(yd_hbm.at[i0b.at[0]], b0.at[0], gs[0][0]),
              pltpu.async_copy(yd_hbm.at[i1b.at[0]], b1.at[0], gs[0][1]))
    for cc in range(_CNCH):
        slot = cc % 2
        for c in gat[cc]:
            c.wait()
        if cc + 1 < _CNCH:
            gat[cc + 1] = (
                pltpu.async_copy(yd_hbm.at[i0b.at[cc + 1]], b0.at[1 - slot],
                                 gs[1 - slot][0]),
                pltpu.async_copy(yd_hbm.at[i1b.at[cc + 1]], b1.at[1 - slot],
                                 gs[1 - slot][1]),
            )
        if cc >= 2:
            wr[cc - 2].wait()

        def row(r, carry):
            for c in range(D // 16):
                sl = pl.ds(c * 16, 16)
                ob[slot, r, sl] = b0[slot, r, sl] + b1[slot, r, sl]
            return carry

        lax.fori_loop(0, _CCH, row, 0)
        wr[cc] = pltpu.async_copy(
            ob.at[slot], out_hbm.at[pl.ds(tok0 + cc * _CCH, _CCH)], wsems[slot])
    wr[_CNCH - 2].wait()
    wr[_CNCH - 1].wait()


def kernel(inputs, Wg, W1, W2):
    sel, wb0, wb1 = _gate(inputs, Wg)
    pos, sa = _routing(sel)
    p2 = pos.reshape(N, K)
    p0, p1 = p2[:, 0], p2[:, 1]

    xd, wd = _dispatch_sc(inputs, wb0, wb1,
                          p0.reshape(N // _DCH, _DCH), p1.reshape(N // _DCH, _DCH))

    yd = _grouped_mlp(sa, xd, W1, W2, wd)

    return _combine_sc(yd, p0.reshape(N // _CCH, _CCH), p1.reshape(N // _CCH, _CCH))
